# baseline (speedup 1.0000x reference)
import jax
import jax.numpy as jnp
import numpy as np
from jax import lax
from jax.experimental import pallas as pl
from jax.experimental.pallas import tpu as pltpu
from jax.experimental.pallas import tpu_sc as plsc

B, S, D = 16, 4096, 128
OBS, C, A = 39, 32, 4
NC, NS, L = 2, 16, 16
NW = NC * NS
S_HALF = S // 2
K = 256
NCH = S_HALF // K
DK = D // L
NEG = np.float32(-1e30)
TINY = np.float32(1e-20)
F32 = jnp.float32


def _body(ld_hbm, conds_hbm, cmask_hbm, names_hbm, nmask_hbm, acts_hbm,
          outs_hbm, we_hbm, out_hbm, xacc_hbm, xms_hbm,
          buf0, buf1, xm_buf, mask_buf, ld_buf, we_buf, names_buf, outs_buf,
          nmask_buf, acc_buf, pacc_buf, ms_buf, pms_buf, o_buf, sem0, sem1):
    cidx = lax.axis_index("c")
    sidx = lax.axis_index("s")
    half = sidx % 2
    b = cidx * (B // NC) + sidx // 2
    w = cidx * NS + sidx
    s0 = half * S_HALF
    lane = lax.iota(jnp.int32, L)

    pltpu.sync_copy(ld_hbm.at[b], ld_buf)
    pltpu.sync_copy(we_hbm, we_buf)
    pltpu.sync_copy(cmask_hbm.at[b, pl.ds(s0, S_HALF)], mask_buf)

    obs = [jnp.zeros((L,), F32) for _ in range(DK)]
    for g in range(3):
        ldv = ld_buf[pl.ds(L * g, L)]
        for j in range(L):
            sc = ldv[j]
            for k in range(DK):
                obs[k] = obs[k] + sc * we_buf[L * g + j, pl.ds(L * k, L)]
    obs = tuple(obs)

    def dma(src_hbm, ch, bufref, sem):
        return pltpu.make_async_copy(
            src_hbm.at[b, pl.ds(s0 + ch * K, K), :], bufref, sem)

    def compute_a(bufref, ch, mm):
        base = ch * K

        def group_a(g, mm_):
            tv = jnp.zeros((L,), F32)
            for j in range(L):
                i = g * L + j
                racc = bufref[i, pl.ds(0, L)] * obs[0]
                for k in range(1, DK):
                    racc = racc + bufref[i, pl.ds(L * k, L)] * obs[k]
                tv = jnp.where(lane == j, jnp.sum(racc), tv)
            mv = mask_buf[pl.ds(base + g * L, L)]
            xm = jnp.where(mv > 0, tv, NEG)
            xm_buf[pl.ds(base + g * L, L)] = xm
            return jnp.maximum(mm_, xm)

        return plsc.parallel_loop(0, K // L, 1, unroll=2, carry=mm)(group_a)

    dma(conds_hbm, 0, buf0, sem0).start()
    dma(conds_hbm, 1, buf1, sem1).start()

    def outer_a(g2, mmax):
        for q, (bufref, sem) in enumerate(((buf0, sem0), (buf1, sem1))):
            ch = 2 * g2 + q
            dma(conds_hbm, ch, bufref, sem).wait()
            mmax = compute_a(bufref, ch, mmax)

            @pl.when(ch + 2 < NCH)
            def _():
                dma(conds_hbm, ch + 2, bufref, sem).start()
        return mmax

    mmax = lax.fori_loop(0, NCH // 2, outer_a, jnp.full((L,), NEG, F32))
    m_splat = jnp.full((L,), jnp.maximum(jnp.max(mmax), np.float32(0.0)), F32)

    def compute_b(bufref, ch, carry):
        base = ch * K

        def group_b(g, car):
            a = list(car[:DK])
            se = car[DK]
            xm = xm_buf[pl.ds(base + g * L, L)]
            mv = mask_buf[pl.ds(base + g * L, L)]
            e = jnp.exp(xm - m_splat) * mv
            se = se + e
            for j in range(L):
                i = g * L + j
                wj = e[j]
                for k in range(DK):
                    a[k] = a[k] + wj * bufref[i, pl.ds(L * k, L)]
            return (*a, se)

        return plsc.parallel_loop(0, K // L, 1, unroll=2, carry=carry)(group_b)

    dma(acts_hbm, 0, buf0, sem0).start()
    dma(acts_hbm, 1, buf1, sem1).start()

    def outer_b(g2, carry):
        for q, (bufref, sem) in enumerate(((buf0, sem0), (buf1, sem1))):
            ch = 2 * g2 + q
            dma(acts_hbm, ch, bufref, sem).wait()
            carry = compute_b(bufref, ch, carry)

            @pl.when(ch + 2 < NCH)
            def _():
                dma(acts_hbm, ch + 2, bufref, sem).start()
        return carry

    init = tuple(jnp.zeros((L,), F32) for _ in range(DK + 1))
    res = lax.fori_loop(0, NCH // 2, outer_b, init)
    accs, sum_e = res[:DK], res[DK]

    for k in range(DK):
        acc_buf[pl.ds(L * k, L)] = accs[k]
    s_splat = jnp.full((L,), jnp.sum(sum_e), F32)
    ms_buf[pl.ds(0, L)] = m_splat
    ms_buf[pl.ds(L, L)] = s_splat
    pltpu.sync_copy(acc_buf, xacc_hbm.at[w])
    pltpu.sync_copy(ms_buf, xms_hbm.at[w])
    plsc.subcore_barrier()

    @pl.when(half == 0)
    def _stage_c():
        pltpu.sync_copy(xacc_hbm.at[w + 1], pacc_buf)
        pltpu.sync_copy(xms_hbm.at[w + 1], pms_buf)
        pltpu.sync_copy(names_hbm.at[b], names_buf)
        pltpu.sync_copy(nmask_hbm.at[b], nmask_buf)
        pltpu.sync_copy(outs_hbm.at[b], outs_buf)
        m1 = pms_buf[pl.ds(0, L)]
        s1 = pms_buf[pl.ds(L, L)]
        mg = jnp.maximum(m_splat, m1)
        r0 = jnp.exp(m_splat - mg)
        r1 = jnp.exp(m1 - mg)
        denom = jnp.maximum(s_splat * r0 + s1 * r1, TINY)
        act = [(accs[k] * r0 + pacc_buf[pl.ds(L * k, L)] * r1) / denom
               for k in range(DK)]

        l0 = jnp.zeros((L,), F32)
        l1 = jnp.zeros((L,), F32)
        for c in range(C):
            lacc = names_buf[c, pl.ds(0, L)] * act[0]
            for k in range(1, DK):
                lacc = lacc + names_buf[c, pl.ds(L * k, L)] * act[k]
            t = jnp.sum(lacc)
            if c < L:
                l0 = jnp.where(lane == c, t, l0)
            else:
                l1 = jnp.where(lane == (c - L), t, l1)

        nm0 = nmask_buf[pl.ds(0, L)]
        nm1 = nmask_buf[pl.ds(L, L)]
        x0 = jnp.where(nm0 > 0, l0, NEG)
        x1 = jnp.where(nm1 > 0, l1, NEG)
        m = jnp.maximum(jnp.maximum(jnp.max(x0), jnp.max(x1)), np.float32(0.0))
        e0 = jnp.exp(x0 - m) * nm0
        e1 = jnp.exp(x1 - m) * nm1
        dn = jnp.maximum(jnp.sum(e0) + jnp.sum(e1), TINY)
        w0 = e0 / dn
        w1 = e1 / dn

        idx4 = lane // 4
        out16 = jnp.zeros((L,), F32)
        for k in range(DK):
            wsrc = w0 if k < DK // 2 else w1
            wo = (4 * k) % L
            wsel = jnp.where(idx4 == 0, wsrc[wo],
                   jnp.where(idx4 == 1, wsrc[wo + 1],
                   jnp.where(idx4 == 2, wsrc[wo + 2], wsrc[wo + 3])))
            out16 = out16 + wsel * outs_buf[pl.ds(L * k, L)]
        r = jnp.zeros((L,), F32)
        for a_i in range(A):
            v = out16[a_i] + out16[4 + a_i] + out16[8 + a_i] + out16[12 + a_i]
            r = jnp.where(lane == a_i, v, r)
        o_buf[...] = r
        pltpu.sync_copy(o_buf, out_hbm.at[b])


@jax.jit
def _cond_agent_sc(ld_pad, conds, cmask_f, names, nmask_f, acts, outs_flat, w_eval):
    mesh = plsc.VectorSubcoreMesh(core_axis_name="c", subcore_axis_name="s",
                                  num_cores=NC, num_subcores=NS)
    f = pl.kernel(
        _body,
        out_type=(jax.ShapeDtypeStruct((B, L), F32),
                  jax.ShapeDtypeStruct((NW, D), F32),
                  jax.ShapeDtypeStruct((NW, 2 * L), F32)),
        mesh=mesh,
        compiler_params=pltpu.CompilerParams(needs_layout_passes=False),
        scratch_types=[
            pltpu.VMEM((K, D), F32),
            pltpu.VMEM((K, D), F32),
            pltpu.VMEM((S_HALF,), F32),
            pltpu.VMEM((S_HALF,), F32),
            pltpu.VMEM((48,), F32),
            pltpu.VMEM((48, D), F32),
            pltpu.VMEM((C, D), F32),
            pltpu.VMEM((C * A,), F32),
            pltpu.VMEM((C,), F32),
            pltpu.VMEM((D,), F32),
            pltpu.VMEM((D,), F32),
            pltpu.VMEM((2 * L,), F32),
            pltpu.VMEM((2 * L,), F32),
            pltpu.VMEM((L,), F32),
            pltpu.SemaphoreType.DMA,
            pltpu.SemaphoreType.DMA,
        ],
    )
    out, _, _ = f(ld_pad, conds, cmask_f, names, nmask_f, acts, outs_flat, w_eval)
    return out


def kernel(low_dim, conds_padded, conds_mask, names_padded, names_mask,
           actions_padded, outputs_padded, W_eval):
    ld_pad = jnp.pad(low_dim, ((0, 0), (0, 48 - OBS)))
    we_pad = jnp.pad(W_eval, ((0, 48 - OBS), (0, 0)))
    cmask_f = conds_mask.astype(jnp.float32)
    nmask_f = names_mask.astype(jnp.float32)
    outs_flat = outputs_padded.reshape(B, C * A)
    out = _cond_agent_sc(ld_pad, conds_padded, cmask_f, names_padded, nmask_f,
                         actions_padded, outs_flat, we_pad)
    return out[:, :A]


# speedup vs baseline: 1.1366x; 1.1366x over previous
import jax
import jax.numpy as jnp
import numpy as np
from jax import lax
from jax.experimental import pallas as pl
from jax.experimental.pallas import tpu as pltpu
from jax.experimental.pallas import tpu_sc as plsc

B, S, D = 16, 4096, 128
OBS, C, A = 39, 32, 4
NC, NS, L = 2, 16, 16
NW = NC * NS
S_HALF = S // 2
K = 256
NCH = S_HALF // K
DK = D // L
NEG = np.float32(-1e30)
TINY = np.float32(1e-20)
F32 = jnp.float32


def _body(ld_hbm, conds_hbm, cmask_hbm, names_hbm, nmask_hbm, acts_hbm,
          outs_hbm, we_hbm, out_hbm, xacc_hbm, xms_hbm,
          buf0, buf1, xm_buf, mask_buf, ld_buf, we_buf, names_buf, outs_buf,
          nmask_buf, acc_buf, pacc_buf, ms_buf, pms_buf, o_buf, sem0, sem1):
    cidx = lax.axis_index("c")
    sidx = lax.axis_index("s")
    half = sidx % 2
    b = cidx * (B // NC) + sidx // 2
    w = cidx * NS + sidx
    s0 = half * S_HALF
    lane = lax.iota(jnp.int32, L)

    pltpu.sync_copy(ld_hbm, ld_buf.at[pl.ds(0, B * OBS)])
    pltpu.sync_copy(we_hbm, we_buf)
    pltpu.sync_copy(cmask_hbm.at[b, pl.ds(s0, S_HALF)], mask_buf)

    zeros_i = jnp.zeros((L,), jnp.int32)

    def obs_step(j, o):
        ldv = ld_buf[pl.ds(OBS * b + j, L)]
        sc = jnp.take(ldv, zeros_i)
        return tuple(o[k] + sc * we_buf[j, pl.ds(L * k, L)] for k in range(DK))

    obs = lax.fori_loop(0, OBS, obs_step,
                        tuple(jnp.zeros((L,), F32) for _ in range(DK)))

    def dma(src_hbm, ch, bufref, sem):
        return pltpu.make_async_copy(
            src_hbm.at[b, pl.ds(s0 + ch * K, K), :], bufref, sem)

    def compute_a(bufref, ch, mm):
        base = ch * K

        def group_a(g, mm_):
            tv = jnp.zeros((L,), F32)
            for j in range(L):
                i = g * L + j
                racc = bufref[i, pl.ds(0, L)] * obs[0]
                for k in range(1, DK):
                    racc = racc + bufref[i, pl.ds(L * k, L)] * obs[k]
                tv = jnp.where(lane == j, jnp.sum(racc), tv)
            mv = mask_buf[pl.ds(base + g * L, L)]
            xm = jnp.where(mv > 0, tv, NEG)
            xm_buf[pl.ds(base + g * L, L)] = xm
            return jnp.maximum(mm_, xm)

        return lax.fori_loop(0, K // L, group_a, mm)

    dma(conds_hbm, 0, buf0, sem0).start()
    dma(conds_hbm, 1, buf1, sem1).start()

    def outer_a(g2, mmax):
        for q, (bufref, sem) in enumerate(((buf0, sem0), (buf1, sem1))):
            ch = 2 * g2 + q
            dma(conds_hbm, ch, bufref, sem).wait()
            mmax = compute_a(bufref, ch, mmax)

            @pl.when(ch + 2 < NCH)
            def _():
                dma(conds_hbm, ch + 2, bufref, sem).start()
        return mmax

    mmax = lax.fori_loop(0, NCH // 2, outer_a, jnp.full((L,), NEG, F32))
    m_splat = jnp.full((L,), jnp.maximum(jnp.max(mmax), np.float32(0.0)), F32)

    def compute_b(bufref, ch, carry):
        base = ch * K

        def group_b(g, car):
            a = list(car[:DK])
            se = car[DK]
            xm = xm_buf[pl.ds(base + g * L, L)]
            mv = mask_buf[pl.ds(base + g * L, L)]
            e = jnp.exp(xm - m_splat) * mv
            se = se + e
            for j in range(L):
                i = g * L + j
                wj = e[j]
                for k in range(DK):
                    a[k] = a[k] + wj * bufref[i, pl.ds(L * k, L)]
            return (*a, se)

        return lax.fori_loop(0, K // L, group_b, carry)

    dma(acts_hbm, 0, buf0, sem0).start()
    dma(acts_hbm, 1, buf1, sem1).start()

    def outer_b(g2, carry):
        for q, (bufref, sem) in enumerate(((buf0, sem0), (buf1, sem1))):
            ch = 2 * g2 + q
            dma(acts_hbm, ch, bufref, sem).wait()
            carry = compute_b(bufref, ch, carry)

            @pl.when(ch + 2 < NCH)
            def _():
                dma(acts_hbm, ch + 2, bufref, sem).start()
        return carry

    init = tuple(jnp.zeros((L,), F32) for _ in range(DK + 1))
    res = lax.fori_loop(0, NCH // 2, outer_b, init)
    accs, sum_e = res[:DK], res[DK]

    for k in range(DK):
        acc_buf[pl.ds(L * k, L)] = accs[k]
    s_splat = jnp.full((L,), jnp.sum(sum_e), F32)
    ms_buf[pl.ds(0, L)] = m_splat
    ms_buf[pl.ds(L, L)] = s_splat
    pltpu.sync_copy(acc_buf, xacc_hbm.at[w])
    pltpu.sync_copy(ms_buf, xms_hbm.at[w])
    plsc.subcore_barrier()

    @pl.when(half == 0)
    def _stage_c():
        pltpu.sync_copy(xacc_hbm.at[w + 1], pacc_buf)
        pltpu.sync_copy(xms_hbm.at[w + 1], pms_buf)
        pltpu.sync_copy(names_hbm.at[b], names_buf)
        pltpu.sync_copy(nmask_hbm.at[b], nmask_buf)
        pltpu.sync_copy(outs_hbm.at[b], outs_buf)
        m1 = pms_buf[pl.ds(0, L)]
        s1 = pms_buf[pl.ds(L, L)]
        mg = jnp.maximum(m_splat, m1)
        r0 = jnp.exp(m_splat - mg)
        r1 = jnp.exp(m1 - mg)
        denom = jnp.maximum(s_splat * r0 + s1 * r1, TINY)
        act = [(accs[k] * r0 + pacc_buf[pl.ds(L * k, L)] * r1) / denom
               for k in range(DK)]

        def logit_step(c, carry):
            l0_, l1_ = carry
            lacc = names_buf[c, pl.ds(0, L)] * act[0]
            for k in range(1, DK):
                lacc = lacc + names_buf[c, pl.ds(L * k, L)] * act[k]
            t = jnp.sum(lacc)
            l0_ = jnp.where(lane == c, t, l0_)
            l1_ = jnp.where(lane == c - L, t, l1_)
            return (l0_, l1_)

        l0, l1 = lax.fori_loop(0, C, logit_step,
                               (jnp.zeros((L,), F32), jnp.zeros((L,), F32)))

        nm0 = nmask_buf[pl.ds(0, L)]
        nm1 = nmask_buf[pl.ds(L, L)]
        x0 = jnp.where(nm0 > 0, l0, NEG)
        x1 = jnp.where(nm1 > 0, l1, NEG)
        m = jnp.maximum(jnp.maximum(jnp.max(x0), jnp.max(x1)), np.float32(0.0))
        e0 = jnp.exp(x0 - m) * nm0
        e1 = jnp.exp(x1 - m) * nm1
        dn = jnp.maximum(jnp.sum(e0) + jnp.sum(e1), TINY)
        w0 = e0 / dn
        w1 = e1 / dn

        idx4 = lane // 4
        out16 = jnp.zeros((L,), F32)
        for k in range(DK):
            wsrc = w0 if k < DK // 2 else w1
            wo = (4 * k) % L
            wsel = jnp.where(idx4 == 0, wsrc[wo],
                   jnp.where(idx4 == 1, wsrc[wo + 1],
                   jnp.where(idx4 == 2, wsrc[wo + 2], wsrc[wo + 3])))
            out16 = out16 + wsel * outs_buf[pl.ds(L * k, L)]
        r = jnp.zeros((L,), F32)
        for a_i in range(A):
            v = out16[a_i] + out16[4 + a_i] + out16[8 + a_i] + out16[12 + a_i]
            r = jnp.where(lane == a_i, v, r)
        o_buf[...] = r
        pltpu.sync_copy(o_buf, out_hbm.at[b])


@jax.jit
def _cond_agent_sc(ld_pad, conds, cmask_f, names, nmask_f, acts, outs_flat, w_eval):
    mesh = plsc.VectorSubcoreMesh(core_axis_name="c", subcore_axis_name="s",
                                  num_cores=NC, num_subcores=NS)
    f = pl.kernel(
        _body,
        out_type=(jax.ShapeDtypeStruct((B, L), F32),
                  jax.ShapeDtypeStruct((NW, D), F32),
                  jax.ShapeDtypeStruct((NW, 2 * L), F32)),
        mesh=mesh,
        compiler_params=pltpu.CompilerParams(needs_layout_passes=False),
        scratch_types=[
            pltpu.VMEM((K, D), F32),
            pltpu.VMEM((K, D), F32),
            pltpu.VMEM((S_HALF,), F32),
            pltpu.VMEM((S_HALF,), F32),
            pltpu.VMEM((B * OBS + L, ), F32),
            pltpu.VMEM((OBS, D), F32),
            pltpu.VMEM((C, D), F32),
            pltpu.VMEM((C * A,), F32),
            pltpu.VMEM((C,), F32),
            pltpu.VMEM((D,), F32),
            pltpu.VMEM((D,), F32),
            pltpu.VMEM((2 * L,), F32),
            pltpu.VMEM((2 * L,), F32),
            pltpu.VMEM((L,), F32),
            pltpu.SemaphoreType.DMA,
            pltpu.SemaphoreType.DMA,
        ],
    )
    out, _, _ = f(ld_pad, conds, cmask_f, names, nmask_f, acts, outs_flat, w_eval)
    return out


def kernel(low_dim, conds_padded, conds_mask, names_padded, names_mask,
           actions_padded, outputs_padded, W_eval):
    ld_flat = low_dim.reshape(B * OBS)
    cmask_f = conds_mask.astype(jnp.float32)
    nmask_f = names_mask.astype(jnp.float32)
    outs_flat = outputs_padded.reshape(B, C * A)
    out = _cond_agent_sc(ld_flat, conds_padded, cmask_f, names_padded, nmask_f,
                         actions_padded, outs_flat, W_eval)
    return out[:, :A]
